# baseline (speedup 1.0000x reference)
import functools

import jax
import jax.numpy as jnp
from jax import lax
from jax.experimental import pallas as pl
from jax.experimental.pallas import tpu as pltpu
from jax.experimental.pallas import tpu_sc as plsc

N = 10000
E = 320000
D_IN = 128
D_H = 64
G = 64

NC = 2
NS = 16
NW = NC * NS
EPW = E // NW
CH = 125
NCH = EPW // CH
NB = 4
RPT = N // NS
ZR = 125

_mesh = lambda: plsc.VectorSubcoreMesh(core_axis_name="c", subcore_axis_name="s")
_SC_PARAMS = pltpu.CompilerParams(use_tc_tiling_on_sc=False)


def _sc_hist(ei4):
    @functools.partial(
        pl.kernel,
        out_type=jax.ShapeDtypeStruct((NW, RPT, 16), jnp.float32),
        mesh=_mesh(),
        compiler_params=_SC_PARAMS,
        scratch_types=[
            pltpu.VMEM((NCH, CH), jnp.int32),
            pltpu.VMEM((CH, 16), jnp.float32),
            pltpu.VMEM((ZR, 16), jnp.float32),
            pltpu.VMEM_SHARED((N, 16), jnp.float32),
            pltpu.SemaphoreType.DMA,
            pltpu.SemaphoreType.DMA,
        ],
    )
    def hist_kernel(ei_hbm, out_hbm, didx, ones_v, zeros_v, hacc, sem, zsem):
        c = lax.axis_index("c")
        s = lax.axis_index("s")
        wid = c * NS + s
        one = jnp.ones((16,), jnp.float32)
        zero = jnp.zeros((16,), jnp.float32)

        pltpu.async_copy(ei_hbm.at[1, wid], didx, sem)

        @pl.loop(0, CH)
        def _(i):
            ones_v[i] = one

        @pl.loop(0, ZR)
        def _(i):
            zeros_v[i] = zero

        for k in range(RPT // ZR):
            pltpu.async_copy(zeros_v, hacc.at[pl.ds(s * RPT + k * ZR, ZR)],
                             zsem)

        pltpu.make_async_copy(ei_hbm.at[1, wid], didx, sem).wait()

        for k in range(RPT // ZR):
            pltpu.make_async_copy(zeros_v,
                                  hacc.at[pl.ds(s * RPT + k * ZR, ZR)],
                                  zsem).wait()

        plsc.subcore_barrier()

        for b in range(NB):
            pltpu.async_copy(ones_v, hacc.at[didx.at[b]], sem, add=True)

        @pl.loop(0, NCH - NB)
        def _(j):
            pltpu.make_async_copy(ones_v, hacc.at[didx.at[j]], sem).wait()
            pltpu.async_copy(ones_v, hacc.at[didx.at[j + NB]], sem, add=True)

        for b in range(NB):
            pltpu.make_async_copy(ones_v, hacc.at[didx.at[b]], sem).wait()

        plsc.subcore_barrier()
        pltpu.sync_copy(hacc.at[pl.ds(s * RPT, RPT)], out_hbm.at[wid])

    return hist_kernel(ei4)


def _sc_agg(hs, ei4, zrow):
    @functools.partial(
        pl.kernel,
        out_type=jax.ShapeDtypeStruct((NW, RPT, D_H), jnp.bfloat16),
        mesh=_mesh(),
        compiler_params=_SC_PARAMS,
        scratch_types=[
            pltpu.VMEM((NCH, CH), jnp.int32),
            pltpu.VMEM((NCH, CH), jnp.int32),
            pltpu.VMEM((NB, CH, D_H), jnp.bfloat16),
            pltpu.VMEM((ZR, D_H), jnp.bfloat16),
            pltpu.VMEM_SHARED((N, D_H), jnp.bfloat16),
        ] + [pltpu.SemaphoreType.DMA] * (2 * NB + 2),
    )
    def agg_kernel(hs_hbm, ei_hbm, z_hbm, out_hbm,
                   sidx, didx, rows, zeros_v, acc, *sems):
        gsem = sems[:NB]
        ssem = sems[NB:2 * NB]
        isem = sems[2 * NB]
        zsem = sems[2 * NB + 1]
        c = lax.axis_index("c")
        s = lax.axis_index("s")
        wid = c * NS + s

        pltpu.async_copy(ei_hbm.at[0, wid], sidx, isem)
        pltpu.async_copy(ei_hbm.at[1, wid], didx, isem)
        pltpu.async_copy(z_hbm, zeros_v, zsem)
        pltpu.make_async_copy(z_hbm, zeros_v, zsem).wait()

        for k in range(RPT // ZR):
            pltpu.async_copy(zeros_v, acc.at[pl.ds(s * RPT + k * ZR, ZR)],
                             zsem)

        pltpu.make_async_copy(ei_hbm.at[0, wid], sidx, isem).wait()
        pltpu.make_async_copy(ei_hbm.at[1, wid], didx, isem).wait()

        for b in range(NB):
            pltpu.async_copy(hs_hbm.at[sidx.at[b]], rows.at[b], gsem[b])

        for k in range(RPT // ZR):
            pltpu.make_async_copy(zeros_v,
                                  acc.at[pl.ds(s * RPT + k * ZR, ZR)],
                                  zsem).wait()

        plsc.subcore_barrier()

        @pl.loop(0, NCH, step=NB)
        def _(j0):
            for b in range(NB):
                j = j0 + b
                pltpu.make_async_copy(
                    hs_hbm.at[sidx.at[j]], rows.at[b], gsem[b]).wait()
                pltpu.async_copy(
                    rows.at[b], acc.at[didx.at[j]], ssem[b], add=True)
                pltpu.make_async_copy(
                    rows.at[b], acc.at[didx.at[j]], ssem[b]).wait()

                @pl.when(j + NB < NCH)
                def _():
                    pltpu.async_copy(
                        hs_hbm.at[sidx.at[j + NB]], rows.at[b], gsem[b])

        plsc.subcore_barrier()
        pltpu.sync_copy(acc.at[pl.ds(s * RPT, RPT)], out_hbm.at[wid])

    return agg_kernel(hs, ei4, zrow)


BM = 2000


def _dot(a, b):
    return lax.dot_general(a, b, (((1,), (0,)), ((), ())),
                           precision=lax.Precision.DEFAULT,
                           preferred_element_type=jnp.float32)


def _tc_matmul_scale(x4, W, hist):
    def body(x_ref, w_ref, h0_ref, h1_ref, hs_ref, dis_ref):
        deg = h0_ref[0, :, 0:1] + h1_ref[0, :, 0:1] + 1.0
        dis = lax.rsqrt(deg)
        dis_ref[0] = dis
        hs_ref[0] = (_dot(x_ref[0], w_ref[...]) * dis).astype(jnp.bfloat16)

    return pl.pallas_call(
        body,
        grid=(NS,),
        in_specs=[pl.BlockSpec((1, RPT, D_IN), lambda i: (i, 0, 0)),
                  pl.BlockSpec((D_IN, D_H), lambda i: (0, 0)),
                  pl.BlockSpec((1, RPT, 16), lambda i: (i, 0, 0)),
                  pl.BlockSpec((1, RPT, 16), lambda i: (i + NS, 0, 0))],
        out_specs=[pl.BlockSpec((1, RPT, D_H), lambda i: (i, 0, 0)),
                   pl.BlockSpec((1, RPT, 1), lambda i: (i, 0, 0))],
        out_shape=[jax.ShapeDtypeStruct((NS, RPT, D_H), jnp.bfloat16),
                   jax.ShapeDtypeStruct((NS, RPT, 1), jnp.float32)],
    )(x4, W, hist, hist)


def _tc_mid(agg, hs, dis, b, W):
    def body(a0_ref, a1_ref, hs_ref, d_ref, b_ref, w_ref, o_ref):
        t = (a0_ref[0].astype(jnp.float32) + a1_ref[0].astype(jnp.float32)
             + hs_ref[0].astype(jnp.float32))
        h = jnp.maximum(d_ref[0] * t + b_ref[...], 0.0)
        o_ref[0] = (_dot(h, w_ref[...]) * d_ref[0]).astype(jnp.bfloat16)

    return pl.pallas_call(
        body,
        grid=(NS,),
        in_specs=[pl.BlockSpec((1, RPT, D_H), lambda i: (i, 0, 0)),
                  pl.BlockSpec((1, RPT, D_H), lambda i: (i + NS, 0, 0)),
                  pl.BlockSpec((1, RPT, D_H), lambda i: (i, 0, 0)),
                  pl.BlockSpec((1, RPT, 1), lambda i: (i, 0, 0)),
                  pl.BlockSpec((1, D_H), lambda i: (0, 0)),
                  pl.BlockSpec((D_H, D_H), lambda i: (0, 0))],
        out_specs=pl.BlockSpec((1, RPT, D_H), lambda i: (i, 0, 0)),
        out_shape=jax.ShapeDtypeStruct((NS, RPT, D_H), jnp.bfloat16),
    )(agg, agg, hs, dis, b.reshape(1, D_H), W)


def _tc_final(agg, hs, dis, b, batch2d, Wh, bh):
    NBLK = NS

    def body(a0_ref, a1_ref, hs_ref, d_ref, b_ref, bt_ref, wh_ref, bh_ref,
             o0_ref, o1_ref, o2_ref, o3_ref, sums_acc, cnt_acc):
        i = pl.program_id(0)
        t = (a0_ref[0].astype(jnp.float32) + a1_ref[0].astype(jnp.float32)
             + hs_ref[0].astype(jnp.float32))
        h = jnp.maximum(d_ref[0] * t + b_ref[...], 0.0)
        gid = lax.broadcasted_iota(jnp.int32, (RPT, G), 1)
        onehot = jnp.where(bt_ref[0] == gid, 1.0, 0.0)
        sums = lax.dot_general(onehot, h, (((0,), (0,)), ((), ())),
                               precision=lax.Precision.DEFAULT,
                               preferred_element_type=jnp.float32)
        counts = lax.dot_general(onehot, jnp.ones((RPT, 1), jnp.float32),
                                 (((0,), (0,)), ((), ())),
                                 precision=lax.Precision.HIGHEST,
                                 preferred_element_type=jnp.float32)

        @pl.when(i == 0)
        def _():
            sums_acc[...] = jnp.zeros_like(sums_acc)
            cnt_acc[...] = jnp.zeros_like(cnt_acc)

        sums_acc[...] += sums
        cnt_acc[...] += counts

        @pl.when(i == NBLK - 1)
        def _():
            gm = sums_acc[...] / jnp.maximum(cnt_acc[...], 1.0)
            z = _dot(gm, wh_ref[...]) + bh_ref[...]
            sig = 1.0 / (1.0 + jnp.exp(-z))
            o0_ref[...] = sig[:, 0:1].reshape(1, G)
            o1_ref[...] = sig[:, 1:2].reshape(1, G)
            o2_ref[...] = sig[:, 2:3].reshape(1, G)
            o3_ref[...] = sig[:, 3:4].reshape(1, G)

    return pl.pallas_call(
        body,
        grid=(NBLK,),
        in_specs=[pl.BlockSpec((1, RPT, D_H), lambda i: (i, 0, 0)),
                  pl.BlockSpec((1, RPT, D_H), lambda i: (i + NS, 0, 0)),
                  pl.BlockSpec((1, RPT, D_H), lambda i: (i, 0, 0)),
                  pl.BlockSpec((1, RPT, 1), lambda i: (i, 0, 0)),
                  pl.BlockSpec((1, D_H), lambda i: (0, 0)),
                  pl.BlockSpec((1, RPT, 1), lambda i: (i, 0, 0)),
                  pl.BlockSpec((D_H, 4), lambda i: (0, 0)),
                  pl.BlockSpec((1, 4), lambda i: (0, 0))],
        out_specs=[pl.BlockSpec((1, G), lambda i: (0, 0))] * 4,
        out_shape=[jax.ShapeDtypeStruct((1, G), jnp.float32)] * 4,
        scratch_shapes=[pltpu.VMEM((G, D_H), jnp.float32),
                        pltpu.VMEM((G, 1), jnp.float32)],
    )(agg, agg, hs, dis, b.reshape(1, D_H), batch2d, Wh, bh)


def kernel(x, edge_index, batch, W1, b1, W2, b2, Wc, bc, Wm, bm, Wk, bk, Wf, bf):
    ei4 = edge_index.reshape(2, NW, NCH, CH)
    x4 = x.reshape(NS, RPT, D_IN)
    bt4 = batch.reshape(NS, RPT, 1)
    zrow = jnp.zeros((ZR, D_H), jnp.bfloat16)

    hist = _sc_hist(ei4)
    hs1, dis = _tc_matmul_scale(x4, W1, hist)
    agg1 = _sc_agg(hs1.reshape(N, D_H), ei4, zrow)
    hs2 = _tc_mid(agg1, hs1, dis, b1, W2)
    agg2 = _sc_agg(hs2.reshape(N, D_H), ei4, zrow)

    Wh = jnp.concatenate([Wc, Wm, Wk, Wf], axis=1)
    bh = jnp.stack([bc[0], bm[0], bk[0], bf[0]]).reshape(1, 4)
    o0, o1, o2, o3 = _tc_final(agg2, hs2, dis, b2, bt4, Wh, bh)
    return (o0.reshape(G), o1.reshape(G), o2.reshape(G), o3.reshape(G))


# speedup vs baseline: 1.1574x; 1.1574x over previous
import functools

import jax
import jax.numpy as jnp
from jax import lax
from jax.experimental import pallas as pl
from jax.experimental.pallas import tpu as pltpu
from jax.experimental.pallas import tpu_sc as plsc

N = 10000
E = 320000
D_IN = 128
D_H = 64
G = 64

NC = 2
NS = 16
NW = NC * NS
EPW = E // NW
CH = 125
NCH = EPW // CH
NB = 4
RPT = N // NS
ZR = 125

_mesh = lambda: plsc.VectorSubcoreMesh(core_axis_name="c", subcore_axis_name="s")
_SC_PARAMS = pltpu.CompilerParams(use_tc_tiling_on_sc=False)


def _sc_hist(ei4):
    @functools.partial(
        pl.kernel,
        out_type=jax.ShapeDtypeStruct((NW, RPT, 16), jnp.float32),
        mesh=_mesh(),
        compiler_params=_SC_PARAMS,
        scratch_types=[
            pltpu.VMEM((NCH, CH), jnp.int32),
            pltpu.VMEM((CH, 16), jnp.float32),
            pltpu.VMEM((ZR, 16), jnp.float32),
            pltpu.VMEM_SHARED((N, 16), jnp.float32),
            pltpu.SemaphoreType.DMA,
            pltpu.SemaphoreType.DMA,
        ],
    )
    def hist_kernel(ei_hbm, out_hbm, didx, ones_v, zeros_v, hacc, sem, zsem):
        c = lax.axis_index("c")
        s = lax.axis_index("s")
        wid = c * NS + s
        one = jnp.ones((16,), jnp.float32)
        zero = jnp.zeros((16,), jnp.float32)

        pltpu.async_copy(ei_hbm.at[1, wid], didx, sem)

        @pl.loop(0, CH)
        def _(i):
            ones_v[i] = one

        @pl.loop(0, ZR)
        def _(i):
            zeros_v[i] = zero

        for k in range(RPT // ZR):
            pltpu.async_copy(zeros_v, hacc.at[pl.ds(s * RPT + k * ZR, ZR)],
                             zsem)

        pltpu.make_async_copy(ei_hbm.at[1, wid], didx, sem).wait()

        for k in range(RPT // ZR):
            pltpu.make_async_copy(zeros_v,
                                  hacc.at[pl.ds(s * RPT + k * ZR, ZR)],
                                  zsem).wait()

        plsc.subcore_barrier()

        for b in range(NB):
            pltpu.async_copy(ones_v, hacc.at[didx.at[b]], sem, add=True)

        @pl.loop(0, NCH - NB)
        def _(j):
            pltpu.make_async_copy(ones_v, hacc.at[didx.at[j]], sem).wait()
            pltpu.async_copy(ones_v, hacc.at[didx.at[j + NB]], sem, add=True)

        for b in range(NB):
            pltpu.make_async_copy(ones_v, hacc.at[didx.at[b]], sem).wait()

        plsc.subcore_barrier()
        pltpu.sync_copy(hacc.at[pl.ds(s * RPT, RPT)], out_hbm.at[wid])

    return hist_kernel(ei4).reshape(NC, N, 16)


def _sc_agg(hs, ei4, zrow):
    @functools.partial(
        pl.kernel,
        out_type=jax.ShapeDtypeStruct((NW, RPT, D_H), jnp.bfloat16),
        mesh=_mesh(),
        compiler_params=_SC_PARAMS,
        scratch_types=[
            pltpu.VMEM((NCH, CH), jnp.int32),
            pltpu.VMEM((NCH, CH), jnp.int32),
            pltpu.VMEM((NB, CH, D_H), jnp.bfloat16),
            pltpu.VMEM((ZR, D_H), jnp.bfloat16),
            pltpu.VMEM_SHARED((N, D_H), jnp.bfloat16),
        ] + [pltpu.SemaphoreType.DMA] * (2 * NB + 2),
    )
    def agg_kernel(hs_hbm, ei_hbm, z_hbm, out_hbm,
                   sidx, didx, rows, zeros_v, acc, *sems):
        gsem = sems[:NB]
        ssem = sems[NB:2 * NB]
        isem = sems[2 * NB]
        zsem = sems[2 * NB + 1]
        c = lax.axis_index("c")
        s = lax.axis_index("s")
        wid = c * NS + s

        pltpu.async_copy(ei_hbm.at[0, wid], sidx, isem)
        pltpu.async_copy(ei_hbm.at[1, wid], didx, isem)
        pltpu.async_copy(z_hbm, zeros_v, zsem)
        pltpu.make_async_copy(z_hbm, zeros_v, zsem).wait()

        for k in range(RPT // ZR):
            pltpu.async_copy(zeros_v, acc.at[pl.ds(s * RPT + k * ZR, ZR)],
                             zsem)

        pltpu.make_async_copy(ei_hbm.at[0, wid], sidx, isem).wait()
        pltpu.make_async_copy(ei_hbm.at[1, wid], didx, isem).wait()

        for b in range(NB):
            pltpu.async_copy(hs_hbm.at[sidx.at[b]], rows.at[b], gsem[b])

        for k in range(RPT // ZR):
            pltpu.make_async_copy(zeros_v,
                                  acc.at[pl.ds(s * RPT + k * ZR, ZR)],
                                  zsem).wait()

        plsc.subcore_barrier()

        @pl.loop(0, NCH, step=NB)
        def _(j0):
            for b in range(NB):
                j = j0 + b
                pltpu.make_async_copy(
                    hs_hbm.at[sidx.at[j]], rows.at[b], gsem[b]).wait()
                pltpu.async_copy(
                    rows.at[b], acc.at[didx.at[j]], ssem[b], add=True)
                pltpu.make_async_copy(
                    rows.at[b], acc.at[didx.at[j]], ssem[b]).wait()

                @pl.when(j + NB < NCH)
                def _():
                    pltpu.async_copy(
                        hs_hbm.at[sidx.at[j + NB]], rows.at[b], gsem[b])

        plsc.subcore_barrier()
        pltpu.sync_copy(acc.at[pl.ds(s * RPT, RPT)], out_hbm.at[wid])

    return agg_kernel(hs, ei4, zrow).reshape(NC, N, D_H)


BM = 2000


def _dot(a, b):
    return lax.dot_general(a, b, (((1,), (0,)), ((), ())),
                           precision=lax.Precision.DEFAULT,
                           preferred_element_type=jnp.float32)


def _tc_matmul_scale(x, W, hist):
    def body(x_ref, w_ref, h_ref, hs_ref, dis_ref):
        deg = h_ref[0, :, 0:1] + h_ref[1, :, 0:1] + 1.0
        dis = lax.rsqrt(deg)
        dis_ref[...] = dis
        hs_ref[...] = (_dot(x_ref[...], w_ref[...]) * dis).astype(jnp.bfloat16)

    return pl.pallas_call(
        body,
        grid=(N // BM,),
        in_specs=[pl.BlockSpec((BM, D_IN), lambda i: (i, 0)),
                  pl.BlockSpec((D_IN, D_H), lambda i: (0, 0)),
                  pl.BlockSpec((2, BM, 16), lambda i: (0, i, 0))],
        out_specs=[pl.BlockSpec((BM, D_H), lambda i: (i, 0)),
                   pl.BlockSpec((BM, 1), lambda i: (i, 0))],
        out_shape=[jax.ShapeDtypeStruct((N, D_H), jnp.bfloat16),
                   jax.ShapeDtypeStruct((N, 1), jnp.float32)],
    )(x, W, hist)


def _tc_mid(agg, hs, dis, b, W):
    def body(a_ref, hs_ref, d_ref, b_ref, w_ref, o_ref):
        t = (a_ref[0].astype(jnp.float32) + a_ref[1].astype(jnp.float32)
             + hs_ref[...].astype(jnp.float32))
        h = jnp.maximum(d_ref[...] * t + b_ref[...], 0.0)
        o_ref[...] = (_dot(h, w_ref[...]) * d_ref[...]).astype(jnp.bfloat16)

    return pl.pallas_call(
        body,
        grid=(N // BM,),
        in_specs=[pl.BlockSpec((2, BM, D_H), lambda i: (0, i, 0)),
                  pl.BlockSpec((BM, D_H), lambda i: (i, 0)),
                  pl.BlockSpec((BM, 1), lambda i: (i, 0)),
                  pl.BlockSpec((1, D_H), lambda i: (0, 0)),
                  pl.BlockSpec((D_H, D_H), lambda i: (0, 0))],
        out_specs=pl.BlockSpec((BM, D_H), lambda i: (i, 0)),
        out_shape=jax.ShapeDtypeStruct((N, D_H), jnp.bfloat16),
    )(agg, hs, dis, b.reshape(1, D_H), W)


def _tc_final(agg, hs, dis, b, batch2d, Wh, bh):
    NBLK = N // BM

    def body(a_ref, hs_ref, d_ref, b_ref, bt_ref, wh_ref, bh_ref,
             o0_ref, o1_ref, o2_ref, o3_ref, sums_acc, cnt_acc):
        i = pl.program_id(0)
        t = (a_ref[0].astype(jnp.float32) + a_ref[1].astype(jnp.float32)
             + hs_ref[...].astype(jnp.float32))
        h = jnp.maximum(d_ref[...] * t + b_ref[...], 0.0)
        gid = lax.broadcasted_iota(jnp.int32, (BM, G), 1)
        onehot = jnp.where(bt_ref[...] == gid, 1.0, 0.0)
        sums = lax.dot_general(onehot, h, (((0,), (0,)), ((), ())),
                               precision=lax.Precision.DEFAULT,
                               preferred_element_type=jnp.float32)
        counts = lax.dot_general(onehot, jnp.ones((BM, 1), jnp.float32),
                                 (((0,), (0,)), ((), ())),
                                 precision=lax.Precision.HIGHEST,
                                 preferred_element_type=jnp.float32)

        @pl.when(i == 0)
        def _():
            sums_acc[...] = jnp.zeros_like(sums_acc)
            cnt_acc[...] = jnp.zeros_like(cnt_acc)

        sums_acc[...] += sums
        cnt_acc[...] += counts

        @pl.when(i == NBLK - 1)
        def _():
            gm = sums_acc[...] / jnp.maximum(cnt_acc[...], 1.0)
            z = _dot(gm, wh_ref[...]) + bh_ref[...]
            sig = 1.0 / (1.0 + jnp.exp(-z))
            o0_ref[...] = sig[:, 0:1].reshape(1, G)
            o1_ref[...] = sig[:, 1:2].reshape(1, G)
            o2_ref[...] = sig[:, 2:3].reshape(1, G)
            o3_ref[...] = sig[:, 3:4].reshape(1, G)

    return pl.pallas_call(
        body,
        grid=(NBLK,),
        in_specs=[pl.BlockSpec((2, BM, D_H), lambda i: (0, i, 0)),
                  pl.BlockSpec((BM, D_H), lambda i: (i, 0)),
                  pl.BlockSpec((BM, 1), lambda i: (i, 0)),
                  pl.BlockSpec((1, D_H), lambda i: (0, 0)),
                  pl.BlockSpec((BM, 1), lambda i: (i, 0)),
                  pl.BlockSpec((D_H, 4), lambda i: (0, 0)),
                  pl.BlockSpec((1, 4), lambda i: (0, 0))],
        out_specs=[pl.BlockSpec((1, G), lambda i: (0, 0))] * 4,
        out_shape=[jax.ShapeDtypeStruct((1, G), jnp.float32)] * 4,
        scratch_shapes=[pltpu.VMEM((G, D_H), jnp.float32),
                        pltpu.VMEM((G, 1), jnp.float32)],
    )(agg, hs, dis, b.reshape(1, D_H), batch2d, Wh, bh)


def kernel(x, edge_index, batch, W1, b1, W2, b2, Wc, bc, Wm, bm, Wk, bk, Wf, bf):
    ei4 = edge_index.reshape(2, NW, NCH, CH)
    zrow = jnp.zeros((ZR, D_H), jnp.bfloat16)

    hist = _sc_hist(ei4)
    hs1, dis = _tc_matmul_scale(x, W1, hist)
    agg1 = _sc_agg(hs1, ei4, zrow)
    hs2 = _tc_mid(agg1, hs1, dis, b1, W2)
    agg2 = _sc_agg(hs2, ei4, zrow)

    Wh = jnp.concatenate([Wc, Wm, Wk, Wf], axis=1)
    bh = jnp.stack([bc[0], bm[0], bk[0], bf[0]]).reshape(1, 4)
    o0, o1, o2, o3 = _tc_final(agg2, hs2, dis, b2, batch.reshape(N, 1),
                               Wh, bh)
    return (o0.reshape(G), o1.reshape(G), o2.reshape(G), o3.reshape(G))
